# baseline (device time: 23570 ns/iter reference)
import jax
import jax.numpy as jnp
from jax import lax
from jax.experimental import pallas as pl
from jax.experimental.pallas import tpu as pltpu

P = 16


def kernel(x):
    m, n = x.shape
    hm = m // 2
    sm = hm // P

    def body(x_hbm, out_hbm, xv, cv, rv,
             y_send, y_recv, x_send, x_recv, ssem, lsem):
        my_x = lax.axis_index("x")
        my_y = lax.axis_index("y")
        y_nbr = (my_x, 1 - my_y)
        x_nbr = (1 - my_x, my_y)

        my_base = my_y * m
        peer_base = (1 - my_y) * m

        c0 = pltpu.make_async_copy(
            x_hbm.at[pl.ds(my_x * hm, sm), :], xv.at[pl.ds(0, sm), :],
            lsem.at[0],
        )
        c0.start()
        c1 = pltpu.make_async_copy(
            x_hbm.at[pl.ds(my_x * hm + sm, hm - sm), :],
            xv.at[pl.ds(sm, hm - sm), :],
            lsem.at[1],
        )
        c1.start()
        c2 = pltpu.make_async_copy(
            x_hbm.at[pl.ds((1 - my_x) * hm, hm), :], xv.at[pl.ds(hm, hm), :],
            lsem.at[2],
        )
        c2.start()

        barrier_sem = pltpu.get_barrier_semaphore()
        for nbr in (y_nbr, x_nbr):
            pl.semaphore_signal(
                barrier_sem, inc=1, device_id=nbr,
                device_id_type=pl.DeviceIdType.MESH,
            )
        pl.semaphore_wait(barrier_sem, 2)

        y_rdmas = []
        for s in range(P):
            if s == 0:
                c0.wait()
            elif s == 1:
                c1.wait()
            cv[pl.ds(s * sm, sm), :] = xv[pl.ds(s * sm, sm), :].astype(
                jnp.bfloat16
            )
            r = pltpu.make_async_remote_copy(
                src_ref=cv.at[pl.ds(s * sm, sm), :],
                dst_ref=rv.at[pl.ds(s * sm, sm), :],
                send_sem=y_send.at[s],
                recv_sem=y_recv.at[s],
                device_id=y_nbr,
                device_id_type=pl.DeviceIdType.MESH,
            )
            r.start()
            y_rdmas.append(r)

        c2.wait()
        cv[pl.ds(hm, hm), :] = xv[pl.ds(hm, hm), :].astype(jnp.bfloat16)
        st1 = pltpu.make_async_copy(
            cv.at[pl.ds(0, hm), :],
            out_hbm.at[pl.ds(my_base + my_x * hm, hm), :],
            lsem.at[3],
        )
        st1.start()
        st2 = pltpu.make_async_copy(
            cv.at[pl.ds(hm, hm), :],
            out_hbm.at[pl.ds(my_base + (1 - my_x) * hm, hm), :],
            lsem.at[4],
        )
        st2.start()

        x_rdmas = []
        stores = []
        for s in range(P):
            y_rdmas[s].wait_recv()
            row = peer_base + my_x * hm + s * sm
            r = pltpu.make_async_remote_copy(
                src_ref=rv.at[pl.ds(s * sm, sm), :],
                dst_ref=out_hbm.at[pl.ds(row, sm), :],
                send_sem=x_send.at[s],
                recv_sem=x_recv.at[s],
                device_id=x_nbr,
                device_id_type=pl.DeviceIdType.MESH,
            )
            r.start()
            x_rdmas.append(r)
            st = pltpu.make_async_copy(
                rv.at[pl.ds(s * sm, sm), :],
                out_hbm.at[pl.ds(row, sm), :],
                ssem.at[s],
            )
            st.start()
            stores.append(st)

        for s in range(P):
            x_rdmas[s].wait_recv()
        for s in range(P):
            y_rdmas[s].wait_send()
            x_rdmas[s].wait_send()
            stores[s].wait()
        st1.wait()
        st2.wait()

    return pl.pallas_call(
        body,
        out_shape=jax.ShapeDtypeStruct((2 * m, n), jnp.bfloat16),
        in_specs=[pl.BlockSpec(memory_space=pl.ANY)],
        out_specs=pl.BlockSpec(memory_space=pl.ANY),
        scratch_shapes=[
            pltpu.VMEM((m, n), jnp.float32),
            pltpu.VMEM((m, n), jnp.bfloat16),
            pltpu.VMEM((hm, n), jnp.bfloat16),
            pltpu.SemaphoreType.DMA((P,)),
            pltpu.SemaphoreType.DMA((P,)),
            pltpu.SemaphoreType.DMA((P,)),
            pltpu.SemaphoreType.DMA((P,)),
            pltpu.SemaphoreType.DMA((P,)),
            pltpu.SemaphoreType.DMA((5,)),
        ],
        compiler_params=pltpu.CompilerParams(collective_id=0),
    )(x)


# device time: 23531 ns/iter; 1.0017x vs baseline; 1.0017x over previous
import jax
import jax.numpy as jnp
from jax import lax
from jax.experimental import pallas as pl
from jax.experimental.pallas import tpu as pltpu

P = 16


def kernel(x):
    m, n = x.shape
    hm = m // 2
    sm = hm // P

    def body(x_hbm, out_hbm, xv, cv, rv,
             y_send, y_recv, x_send, x_recv, ssem, lsem):
        my_x = lax.axis_index("x")
        my_y = lax.axis_index("y")
        y_nbr = (my_x, 1 - my_y)
        x_nbr = (1 - my_x, my_y)

        my_base = my_y * m
        peer_base = (1 - my_y) * m

        c0 = pltpu.make_async_copy(
            x_hbm.at[pl.ds(my_x * hm, sm), :], xv.at[pl.ds(0, sm), :],
            lsem.at[0],
        )
        c0.start()
        c1 = pltpu.make_async_copy(
            x_hbm.at[pl.ds(my_x * hm + sm, hm - sm), :],
            xv.at[pl.ds(sm, hm - sm), :],
            lsem.at[1],
        )
        c1.start()
        c2 = pltpu.make_async_copy(
            x_hbm.at[pl.ds((1 - my_x) * hm, hm), :], xv.at[pl.ds(hm, hm), :],
            lsem.at[2],
        )
        c2.start()

        barrier_sem = pltpu.get_barrier_semaphore()
        for nbr in (y_nbr, x_nbr):
            pl.semaphore_signal(
                barrier_sem, inc=1, device_id=nbr,
                device_id_type=pl.DeviceIdType.MESH,
            )
        pl.semaphore_wait(barrier_sem, 2)

        y_rdmas = []
        for s in range(P):
            if s == 0:
                c0.wait()
            elif s == 1:
                c1.wait()
            cv[pl.ds(s * sm, sm), :] = xv[pl.ds(s * sm, sm), :].astype(
                jnp.bfloat16
            )
            r = pltpu.make_async_remote_copy(
                src_ref=cv.at[pl.ds(s * sm, sm), :],
                dst_ref=rv.at[pl.ds(s * sm, sm), :],
                send_sem=y_send.at[s],
                recv_sem=y_recv.at[s],
                device_id=y_nbr,
                device_id_type=pl.DeviceIdType.MESH,
            )
            r.start()
            y_rdmas.append(r)

        c2.wait()
        cv[pl.ds(hm, hm), :] = xv[pl.ds(hm, hm), :].astype(jnp.bfloat16)
        st1 = pltpu.make_async_copy(
            cv.at[pl.ds(0, hm), :],
            out_hbm.at[pl.ds(my_base + my_x * hm, hm), :],
            lsem.at[3],
        )
        st1.start()
        st2 = pltpu.make_async_copy(
            cv.at[pl.ds(hm, hm), :],
            out_hbm.at[pl.ds(my_base + (1 - my_x) * hm, hm), :],
            lsem.at[4],
        )
        st2.start()

        x_rdmas = []
        stores = []
        for s in range(P):
            y_rdmas[s].wait_recv()
            row = peer_base + my_x * hm + s * sm
            r = pltpu.make_async_remote_copy(
                src_ref=rv.at[pl.ds(s * sm, sm), :],
                dst_ref=out_hbm.at[pl.ds(row, sm), :],
                send_sem=x_send.at[s],
                recv_sem=x_recv.at[s],
                device_id=x_nbr,
                device_id_type=pl.DeviceIdType.MESH,
            )
            r.start()
            x_rdmas.append(r)
            st = pltpu.make_async_copy(
                rv.at[pl.ds(s * sm, sm), :],
                out_hbm.at[pl.ds(row, sm), :],
                ssem.at[s],
            )
            st.start()
            stores.append(st)

        for s in range(P):
            x_rdmas[s].wait_recv()
        for s in range(P):
            y_rdmas[s].wait_send()
            x_rdmas[s].wait_send()
            stores[s].wait()
        st1.wait()
        st2.wait()

    return pl.pallas_call(
        body,
        out_shape=jax.ShapeDtypeStruct((2 * m, n), jnp.bfloat16),
        in_specs=[pl.BlockSpec(memory_space=pltpu.MemorySpace.HBM)],
        out_specs=pl.BlockSpec(memory_space=pltpu.MemorySpace.HBM),
        scratch_shapes=[
            pltpu.VMEM((m, n), jnp.float32),
            pltpu.VMEM((m, n), jnp.bfloat16),
            pltpu.VMEM((hm, n), jnp.bfloat16),
            pltpu.SemaphoreType.DMA((P,)),
            pltpu.SemaphoreType.DMA((P,)),
            pltpu.SemaphoreType.DMA((P,)),
            pltpu.SemaphoreType.DMA((P,)),
            pltpu.SemaphoreType.DMA((P,)),
            pltpu.SemaphoreType.DMA((5,)),
        ],
        compiler_params=pltpu.CompilerParams(collective_id=0),
    )(x)


# device time: 22918 ns/iter; 1.0284x vs baseline; 1.0267x over previous
import jax
import jax.numpy as jnp
from jax import lax
from jax.experimental import pallas as pl
from jax.experimental.pallas import tpu as pltpu

SLICES = [64] * 15 + [32, 32]
P = len(SLICES)


def kernel(x):
    m, n = x.shape
    hm = m // 2
    assert sum(SLICES) == hm

    def body(x_ref, out_ref, cv, rv, y_send, y_recv, x_send, x_recv,
             ssem, lsem):
        my_x = lax.axis_index("x")
        my_y = lax.axis_index("y")
        y_nbr = (my_x, 1 - my_y)
        x_nbr = (1 - my_x, my_y)

        my_base = my_y * m
        peer_base = (1 - my_y) * m

        barrier_sem = pltpu.get_barrier_semaphore()
        for nbr in (y_nbr, x_nbr):
            pl.semaphore_signal(
                barrier_sem, inc=1, device_id=nbr,
                device_id_type=pl.DeviceIdType.MESH,
            )
        pl.semaphore_wait(barrier_sem, 2)

        y_rdmas = []
        off = 0
        for s, sz in enumerate(SLICES):
            cv[pl.ds(off, sz), :] = x_ref[
                pl.ds(my_x * hm + off, sz), :
            ].astype(jnp.bfloat16)
            r = pltpu.make_async_remote_copy(
                src_ref=cv.at[pl.ds(off, sz), :],
                dst_ref=rv.at[pl.ds(off, sz), :],
                send_sem=y_send.at[s],
                recv_sem=y_recv.at[s],
                device_id=y_nbr,
                device_id_type=pl.DeviceIdType.MESH,
            )
            r.start()
            y_rdmas.append(r)
            off += sz

        out_ref[pl.ds(my_base + my_x * hm, hm), :] = cv[...]
        out_ref[pl.ds(my_base + (1 - my_x) * hm, hm), :] = x_ref[
            pl.ds((1 - my_x) * hm, hm), :
        ].astype(jnp.bfloat16)

        x_rdmas = []
        stores = []
        off = 0
        for s, sz in enumerate(SLICES):
            y_rdmas[s].wait_recv()
            row = peer_base + my_x * hm + off
            r = pltpu.make_async_remote_copy(
                src_ref=rv.at[pl.ds(off, sz), :],
                dst_ref=out_ref.at[pl.ds(row, sz), :],
                send_sem=x_send.at[s],
                recv_sem=x_recv.at[s],
                device_id=x_nbr,
                device_id_type=pl.DeviceIdType.MESH,
            )
            r.start()
            x_rdmas.append(r)
            st = pltpu.make_async_copy(
                rv.at[pl.ds(off, sz), :],
                out_ref.at[pl.ds(row, sz), :],
                ssem.at[s],
            )
            st.start()
            stores.append(st)
            off += sz

        for s in range(P):
            x_rdmas[s].wait_recv()
        for s in range(P):
            y_rdmas[s].wait_send()
            x_rdmas[s].wait_send()
            stores[s].wait()

    return pl.pallas_call(
        body,
        out_shape=jax.ShapeDtypeStruct((2 * m, n), jnp.bfloat16),
        in_specs=[pl.BlockSpec(memory_space=pltpu.VMEM)],
        out_specs=pl.BlockSpec(memory_space=pltpu.VMEM),
        scratch_shapes=[
            pltpu.VMEM((m // 2, n), jnp.bfloat16),
            pltpu.VMEM((m // 2, n), jnp.bfloat16),
            pltpu.SemaphoreType.DMA((P,)),
            pltpu.SemaphoreType.DMA((P,)),
            pltpu.SemaphoreType.DMA((P,)),
            pltpu.SemaphoreType.DMA((P,)),
            pltpu.SemaphoreType.DMA((P,)),
            pltpu.SemaphoreType.DMA((2,)),
        ],
        compiler_params=pltpu.CompilerParams(collective_id=0),
    )(x)
